# baseline (speedup 1.0000x reference)
import functools

import jax
import jax.numpy as jnp
from jax.experimental import pallas as pl
from jax.experimental.pallas import tpu as pltpu


def _gating_kernel(nblk, x_ref, w1b_ref, b1_ref, w2_ref, b2_ref, o_ref,
                   w2b_ref, h_ref):
    i = pl.program_id(0)

    @pl.when(i == 0)
    def _cast_weights():
        w2b_ref[...] = w2_ref[...].astype(jnp.bfloat16)

    slot_b = jax.lax.rem(i + 1, 2)
    hp_raw = h_ref[pl.ds(slot_b, 1), :, :][0]
    hp = jnp.maximum(hp_raw + b1_ref[...], 0.0).astype(jnp.bfloat16)
    logits = jnp.dot(hp, w2b_ref[...], preferred_element_type=jnp.float32)
    logits = logits + b2_ref[...]
    m = jnp.max(logits, axis=-1, keepdims=True)
    e = jnp.exp(logits - m)
    o_ref[...] = e * (1.0 / jnp.sum(e, axis=-1, keepdims=True))

    d_model = x_ref.shape[1]
    nk = 4
    ck = d_model // nk
    h = None
    for k in range(nk):
        xb = x_ref[:, k * ck:(k + 1) * ck].astype(jnp.bfloat16)
        p = jnp.dot(xb, w1b_ref[k * ck:(k + 1) * ck, :],
                    preferred_element_type=jnp.float32)
        h = p if h is None else h + p
    slot_a = jax.lax.rem(i, 2)
    h_ref[pl.ds(slot_a, 1), :, :] = h[None]


def kernel(x, W1, b1, W2, b2):
    tokens, d_model = x.shape
    hidden, n_experts = W2.shape
    bt = 1024
    nblk = tokens // bt
    w1b = W1.astype(jnp.bfloat16)
    b1r = b1.reshape(1, hidden)
    b2r = b2.reshape(1, n_experts)
    body = functools.partial(_gating_kernel, nblk)
    return pl.pallas_call(
        body,
        grid=(nblk + 1,),
        in_specs=[
            pl.BlockSpec((bt, d_model), lambda i: (jnp.minimum(i, nblk - 1), 0)),
            pl.BlockSpec((d_model, hidden), lambda i: (0, 0)),
            pl.BlockSpec((1, hidden), lambda i: (0, 0)),
            pl.BlockSpec((hidden, n_experts), lambda i: (0, 0)),
            pl.BlockSpec((1, n_experts), lambda i: (0, 0)),
        ],
        out_specs=pl.BlockSpec((bt, n_experts),
                               lambda i: (jnp.maximum(i - 1, 0), 0)),
        out_shape=jax.ShapeDtypeStruct((tokens, n_experts), jnp.float32),
        scratch_shapes=[
            pltpu.VMEM((hidden, n_experts), jnp.bfloat16),
            pltpu.VMEM((2, bt, hidden), jnp.float32),
        ],
        compiler_params=pltpu.CompilerParams(
            dimension_semantics=("arbitrary",),
        ),
    )(x, w1b, b1r, W2, b2r)


# speedup vs baseline: 1.1274x; 1.1274x over previous
import functools

import jax
import jax.numpy as jnp
from jax.experimental import pallas as pl
from jax.experimental.pallas import tpu as pltpu


def _gating_kernel(nblk, x_ref, w1_ref, b1_ref, w2_ref, b2_ref, o_ref,
                   w1b_ref, w2b_ref, h_ref):
    i = pl.program_id(0)

    @pl.when(i == 0)
    def _cast_weights():
        w1b_ref[...] = w1_ref[...].astype(jnp.bfloat16)
        w2b_ref[...] = w2_ref[...].astype(jnp.bfloat16)

    d_model = x_ref.shape[1]
    nk = 4
    ck = d_model // nk

    def chunk_dot(k):
        xb = x_ref[:, k * ck:(k + 1) * ck].astype(jnp.bfloat16)
        return jnp.dot(xb, w1b_ref[k * ck:(k + 1) * ck, :],
                       preferred_element_type=jnp.float32)

    h = chunk_dot(0)

    slot_b = jax.lax.rem(i + 1, 2)
    hp_raw = h_ref[pl.ds(slot_b, 1), :, :][0]
    hp = jnp.maximum(hp_raw + b1_ref[...], 0.0).astype(jnp.bfloat16)
    logits = jnp.dot(hp, w2b_ref[...], preferred_element_type=jnp.float32)
    logits = logits + b2_ref[...]
    m = jnp.max(logits, axis=-1, keepdims=True)
    e = jnp.exp(logits - m)
    o_ref[...] = e * (1.0 / jnp.sum(e, axis=-1, keepdims=True))

    for k in range(1, nk):
        h = h + chunk_dot(k)
    slot_a = jax.lax.rem(i, 2)
    h_ref[pl.ds(slot_a, 1), :, :] = h[None]


def kernel(x, W1, b1, W2, b2):
    tokens, d_model = x.shape
    hidden, n_experts = W2.shape
    bt = 512
    nblk = tokens // bt
    b1r = b1.reshape(1, hidden)
    b2r = b2.reshape(1, n_experts)
    body = functools.partial(_gating_kernel, nblk)
    return pl.pallas_call(
        body,
        grid=(nblk + 1,),
        in_specs=[
            pl.BlockSpec((bt, d_model), lambda i: (jnp.minimum(i, nblk - 1), 0)),
            pl.BlockSpec((d_model, hidden), lambda i: (0, 0)),
            pl.BlockSpec((1, hidden), lambda i: (0, 0)),
            pl.BlockSpec((hidden, n_experts), lambda i: (0, 0)),
            pl.BlockSpec((1, n_experts), lambda i: (0, 0)),
        ],
        out_specs=pl.BlockSpec((bt, n_experts),
                               lambda i: (jnp.maximum(i - 1, 0), 0)),
        out_shape=jax.ShapeDtypeStruct((tokens, n_experts), jnp.float32),
        scratch_shapes=[
            pltpu.VMEM((d_model, hidden), jnp.bfloat16),
            pltpu.VMEM((hidden, n_experts), jnp.bfloat16),
            pltpu.VMEM((2, bt, hidden), jnp.float32),
        ],
        compiler_params=pltpu.CompilerParams(
            dimension_semantics=("arbitrary",),
        ),
    )(x, W1, b1r, W2, b2r)


# speedup vs baseline: 1.1635x; 1.0320x over previous
import functools

import jax
import jax.numpy as jnp
from jax.experimental import pallas as pl
from jax.experimental.pallas import tpu as pltpu


def _tail(hp_raw, b1_ref, w2b_ref, b2_ref):
    hp = jnp.maximum(hp_raw + b1_ref[...], 0.0).astype(jnp.bfloat16)
    logits = jnp.dot(hp, w2b_ref[...], preferred_element_type=jnp.float32)
    logits = logits + b2_ref[...]
    m = jnp.max(logits, axis=-1, keepdims=True)
    e = jnp.exp(logits - m)
    return e * (1.0 / jnp.sum(e, axis=-1, keepdims=True))


def _gating_kernel(nblk, x_ref, w1_ref, b1_ref, w2_ref, b2_ref, o_ref,
                   w1b_ref, w2b_ref, h_ref):
    i = pl.program_id(0)
    bt = x_ref.shape[0]

    @pl.when(i == 0)
    def _cast_weights():
        w1b_ref[...] = w1_ref[...].astype(jnp.bfloat16)
        w2b_ref[...] = w2_ref[...].astype(jnp.bfloat16)

    slot_b = jax.lax.rem(i + 1, 2)
    hp_raw = h_ref[pl.ds(slot_b, 1), :, :][0]
    o_ref[pl.ds(slot_b * bt, bt), :] = _tail(hp_raw, b1_ref, w2b_ref, b2_ref)

    d_model = x_ref.shape[1]
    nk = 4
    ck = d_model // nk
    h = None
    for k in range(nk):
        xb = x_ref[:, k * ck:(k + 1) * ck].astype(jnp.bfloat16)
        p = jnp.dot(xb, w1b_ref[k * ck:(k + 1) * ck, :],
                    preferred_element_type=jnp.float32)
        h = p if h is None else h + p
    slot_a = jax.lax.rem(i, 2)
    h_ref[pl.ds(slot_a, 1), :, :] = h[None]

    @pl.when(i == nblk - 1)
    def _last_tail():
        o_ref[pl.ds(slot_a * bt, bt), :] = _tail(h, b1_ref, w2b_ref, b2_ref)


def kernel(x, W1, b1, W2, b2):
    tokens, d_model = x.shape
    hidden, n_experts = W2.shape
    bt = 512
    nblk = tokens // bt
    b1r = b1.reshape(1, hidden)
    b2r = b2.reshape(1, n_experts)
    body = functools.partial(_gating_kernel, nblk)
    return pl.pallas_call(
        body,
        grid=(nblk,),
        in_specs=[
            pl.BlockSpec((bt, d_model), lambda i: (i, 0)),
            pl.BlockSpec((d_model, hidden), lambda i: (0, 0)),
            pl.BlockSpec((1, hidden), lambda i: (0, 0)),
            pl.BlockSpec((hidden, n_experts), lambda i: (0, 0)),
            pl.BlockSpec((1, n_experts), lambda i: (0, 0)),
        ],
        out_specs=pl.BlockSpec((2 * bt, n_experts),
                               lambda i: (jnp.maximum(i - 1, 0) // 2, 0)),
        out_shape=jax.ShapeDtypeStruct((tokens, n_experts), jnp.float32),
        scratch_shapes=[
            pltpu.VMEM((d_model, hidden), jnp.bfloat16),
            pltpu.VMEM((hidden, n_experts), jnp.bfloat16),
            pltpu.VMEM((2, bt, hidden), jnp.float32),
        ],
        compiler_params=pltpu.CompilerParams(
            dimension_semantics=("arbitrary",),
        ),
    )(x, W1, b1r, W2, b2r)
